# baseline (speedup 1.0000x reference)
import functools

import jax
import jax.numpy as jnp
from jax import lax
from jax.experimental import pallas as pl
from jax.experimental.pallas import tpu as pltpu
from jax.experimental.pallas import tpu_sc as plsc

D = 64
BATCH, SEQ = 4096, 50
NC, NS = 2, 16
NW = NC * NS
BW = BATCH // NW
ET, ES = D // 8, 8
BT = BATCH // 128


def _build():
  mesh = plsc.VectorSubcoreMesh(core_axis_name="c", subcore_axis_name="s")

  @functools.partial(
      pl.kernel,
      mesh=mesh,
      compiler_params=pltpu.CompilerParams(
          use_tc_tiling_on_sc=False, needs_layout_passes=False),
      out_type=jax.ShapeDtypeStruct((SEQ, ET, BT, ES, BW), jnp.float32),
      scratch_types=[
          pltpu.VMEM((SEQ, BW), jnp.int32),
          pltpu.VMEM((2, BW, D), jnp.float32),
          pltpu.VMEM((2, ET, ES, BW), jnp.float32),
          pltpu.SemaphoreType.DMA,
          pltpu.SemaphoreType.DMA,
          pltpu.SemaphoreType.DMA,
          pltpu.SemaphoreType.DMA,
      ],
  )
  def emb(idx_hbm, table_hbm, out_hbm, idx_v, rows_v, tblk_v,
          gsem0, gsem1, osem0, osem1):
    wid = lax.axis_index("s") * NC + lax.axis_index("c")
    col = pl.multiple_of(wid * BW, BW)
    pltpu.sync_copy(idx_hbm.at[:, pl.ds(col, BW)], idx_v)

    gsems = (gsem0, gsem1)
    osems = (osem0, osem1)
    lane = lax.iota(jnp.int32, 16)
    row_ids = [lane + 16 * g for g in range(8)]

    def start_gather(s, b):
      pltpu.make_async_copy(
          table_hbm.at[idx_v.at[s]], rows_v.at[b], gsems[b]).start()

    def wait_gather(s, b):
      pltpu.make_async_copy(
          table_hbm.at[idx_v.at[s]], rows_v.at[b], gsems[b]).wait()

    def transpose(b):
      rows = rows_v.at[b]
      tblk = tblk_v.at[b]

      def tr_body(tr, carry):
        for e_sub in range(ES):
          e = tr * ES + e_sub
          col_ids = jnp.full((16,), 0, jnp.int32) + e
          for g in range(8):
            v = plsc.load_gather(rows, [row_ids[g], col_ids])
            tblk[tr, e_sub, pl.ds(16 * g, 16)] = v
        return carry

      lax.fori_loop(0, ET, tr_body, 0)

    def start_out(s, b):
      pltpu.make_async_copy(
          tblk_v.at[b], out_hbm.at[s, :, wid], osems[b]).start()

    def wait_out(s, b):
      pltpu.make_async_copy(
          tblk_v.at[b], out_hbm.at[s, :, wid], osems[b]).wait()

    start_gather(0, 0)
    start_gather(1, 1)

    def body(i, carry):
      for b in range(2):
        s = 2 * i + b
        wait_gather(s, b)

        @pl.when(i >= 1)
        def _():
          wait_out(s, b)

        transpose(b)

        @pl.when(s + 2 < SEQ)
        def _():
          start_gather(s + 2, b)

        start_out(s, b)
      return carry

    lax.fori_loop(0, SEQ // 2, body, 0)
    wait_out(SEQ - 2, 0)
    wait_out(SEQ - 1, 1)

  return emb


_emb = _build()


def kernel(item_inputs, table):
  idx_t = item_inputs.T.astype(jnp.int32)
  out5 = _emb(idx_t, table)
  out = out5.transpose(2, 4, 0, 1, 3).reshape(BATCH, SEQ, D)
  return out


# speedup vs baseline: 1.1989x; 1.1989x over previous
import functools

import jax
import jax.numpy as jnp
from jax import lax
from jax.experimental import pallas as pl
from jax.experimental.pallas import tpu as pltpu
from jax.experimental.pallas import tpu_sc as plsc

D = 64
BATCH, SEQ = 4096, 50
NC, NS = 2, 16
NW = NC * NS
BW = BATCH // NW
ET, ES = D // 8, 8
BT = BATCH // 128
BLK = ES * BW


def _build():
  mesh = plsc.VectorSubcoreMesh(core_axis_name="c", subcore_axis_name="s")

  @functools.partial(
      pl.kernel,
      mesh=mesh,
      compiler_params=pltpu.CompilerParams(
          use_tc_tiling_on_sc=False, needs_layout_passes=False),
      out_type=jax.ShapeDtypeStruct((SEQ, ET, BT, BLK), jnp.float32),
      scratch_types=[
          pltpu.VMEM((SEQ, BW), jnp.int32),
          pltpu.VMEM((2, BW, D), jnp.float32),
          pltpu.VMEM((2, ET, BLK), jnp.float32),
          pltpu.SemaphoreType.DMA,
          pltpu.SemaphoreType.DMA,
          pltpu.SemaphoreType.DMA,
          pltpu.SemaphoreType.DMA,
      ],
  )
  def emb(idx_hbm, table_hbm, out_hbm, idx_v, rows_v, tblk_v,
          gsem0, gsem1, osem0, osem1):
    wid = lax.axis_index("s") * NC + lax.axis_index("c")
    col = pl.multiple_of(wid * BW, BW)
    pltpu.sync_copy(idx_hbm.at[:, pl.ds(col, BW)], idx_v)

    gsems = (gsem0, gsem1)
    osems = (osem0, osem1)
    lane = lax.iota(jnp.int32, 16)
    row_ids = [lane + 16 * g for g in range(8)]

    def start_gather(s, b):
      pltpu.make_async_copy(
          table_hbm.at[idx_v.at[s]], rows_v.at[b], gsems[b]).start()

    def wait_gather(s, b):
      pltpu.make_async_copy(
          table_hbm.at[idx_v.at[s]], rows_v.at[b], gsems[b]).wait()

    def transpose(b):
      rows = rows_v.at[b]
      tblk = tblk_v.at[b]

      def tr_body(tr, carry):
        for half in range(2):
          vs = []
          for e_sub in range(4 * half, 4 * half + 4):
            e_vec = jnp.zeros((16,), jnp.int32) + (tr * ES + e_sub)
            for g in range(8):
              vs.append(plsc.load_gather(rows, [row_ids[g], e_vec]))
          i = 0
          for e_sub in range(4 * half, 4 * half + 4):
            for g in range(8):
              tblk[tr, pl.ds(e_sub * BW + 16 * g, 16)] = vs[i]
              i += 1
        return carry

      lax.fori_loop(0, ET, tr_body, 0)

    def start_out(s, b):
      pltpu.make_async_copy(
          tblk_v.at[b], out_hbm.at[s, :, wid], osems[b]).start()

    def wait_out(s, b):
      pltpu.make_async_copy(
          tblk_v.at[b], out_hbm.at[s, :, wid], osems[b]).wait()

    start_gather(0, 0)
    start_gather(1, 1)

    def body(i, carry):
      for b in range(2):
        s = 2 * i + b
        wait_gather(s, b)

        @pl.when(i >= 1)
        def _():
          wait_out(s, b)

        transpose(b)

        @pl.when(s + 2 < SEQ)
        def _():
          start_gather(s + 2, b)

        start_out(s, b)
      return carry

    lax.fori_loop(0, SEQ // 2, body, 0)
    wait_out(SEQ - 2, 0)
    wait_out(SEQ - 1, 1)

  return emb


_emb = _build()


def kernel(item_inputs, table):
  idx_t = item_inputs.T.astype(jnp.int32)
  out4 = _emb(idx_t, table)
  out = out4.reshape(SEQ, ET, BT, ES, BW).transpose(2, 4, 0, 1, 3)
  return out.reshape(BATCH, SEQ, D)


# speedup vs baseline: 2.8936x; 2.4135x over previous
import functools

import jax
import jax.numpy as jnp
from jax import lax
from jax.experimental import pallas as pl
from jax.experimental.pallas import tpu as pltpu
from jax.experimental.pallas import tpu_sc as plsc

D = 64
BATCH, SEQ = 4096, 50
NC, NS = 2, 16
NW = NC * NS
BW = BATCH // NW
ET, ES = D // 8, 8
BT = BATCH // 128
BLK = ES * BW


def _build():
  mesh = plsc.VectorSubcoreMesh(core_axis_name="c", subcore_axis_name="s")

  @functools.partial(
      pl.kernel,
      mesh=mesh,
      compiler_params=pltpu.CompilerParams(
          use_tc_tiling_on_sc=False, needs_layout_passes=False),
      out_type=jax.ShapeDtypeStruct((SEQ, ET, BT, BLK), jnp.float32),
      scratch_types=[
          pltpu.VMEM((SEQ, BW), jnp.int32),
          pltpu.VMEM((2, BW, D), jnp.float32),
          pltpu.VMEM((2, ET * BLK), jnp.float32),
          pltpu.SemaphoreType.DMA,
          pltpu.SemaphoreType.DMA,
          pltpu.SemaphoreType.DMA,
          pltpu.SemaphoreType.DMA,
      ],
  )
  def emb(idx_hbm, table_hbm, out_hbm, idx_v, rows_v, tblk_v,
          gsem0, gsem1, osem0, osem1):
    wid = lax.axis_index("s") * NC + lax.axis_index("c")
    col = pl.multiple_of(wid * BW, BW)
    pltpu.sync_copy(idx_hbm.at[:, pl.ds(col, BW)], idx_v)

    gsems = (gsem0, gsem1)
    osems = (osem0, osem1)
    lane = lax.iota(jnp.int32, 16)
    qs = [(lane + j) & 15 for j in range(16)]
    st_base = [qs[j] * BW + lane for j in range(16)]

    def start_gather(s, b):
      pltpu.make_async_copy(
          table_hbm.at[idx_v.at[s]], rows_v.at[b], gsems[b]).start()

    def wait_gather(s, b):
      pltpu.make_async_copy(
          table_hbm.at[idx_v.at[s]], rows_v.at[b], gsems[b]).wait()

    def transpose(b):
      rows = rows_v.at[b]
      tblk = tblk_v.at[b]

      def g_body(g, carry):
        row_idx = lane + g * 16
        for c in range(D // 16):
          st_off = 16 * c * BW + 16 * g
          vs = [
              plsc.load_gather(rows, [row_idx, qs[j] + 16 * c])
              for j in range(16)
          ]
          for j in range(16):
            plsc.store_scatter(tblk, [st_base[j] + st_off], vs[j])
        return carry

      lax.fori_loop(0, BW // 16, g_body, 0)

    def start_out(s, b):
      for tr in range(ET):
        pltpu.make_async_copy(
            tblk_v.at[b, pl.ds(tr * BLK, BLK)],
            out_hbm.at[s, tr, wid],
            osems[b]).start()

    def wait_out(s, b):
      for tr in range(ET):
        pltpu.make_async_copy(
            tblk_v.at[b, pl.ds(tr * BLK, BLK)],
            out_hbm.at[s, tr, wid],
            osems[b]).wait()

    start_gather(0, 0)
    start_gather(1, 1)

    def body(i, carry):
      for b in range(2):
        s = 2 * i + b
        wait_gather(s, b)

        @pl.when(i >= 1)
        def _():
          wait_out(s, b)

        transpose(b)

        @pl.when(s + 2 < SEQ)
        def _():
          start_gather(s + 2, b)

        start_out(s, b)
      return carry

    lax.fori_loop(0, SEQ // 2, body, 0)
    wait_out(SEQ - 2, 0)
    wait_out(SEQ - 1, 1)

  return emb


_emb = _build()


def kernel(item_inputs, table):
  idx_t = item_inputs.T.astype(jnp.int32)
  out4 = _emb(idx_t, table)
  out = out4.reshape(SEQ, ET, BT, ES, BW).transpose(2, 4, 0, 1, 3)
  return out.reshape(BATCH, SEQ, D)


# speedup vs baseline: 3.1568x; 1.0909x over previous
import functools

import jax
import jax.numpy as jnp
from jax import lax
from jax.experimental import pallas as pl
from jax.experimental.pallas import tpu as pltpu
from jax.experimental.pallas import tpu_sc as plsc

D = 64
BATCH, SEQ = 4096, 50
NC, NS = 2, 16
NW = NC * NS
BW = BATCH // NW
ET, ES = D // 8, 8
BT = BATCH // 128
BLK = ES * BW


def _build():
  mesh = plsc.VectorSubcoreMesh(core_axis_name="c", subcore_axis_name="s")

  @functools.partial(
      pl.kernel,
      mesh=mesh,
      compiler_params=pltpu.CompilerParams(
          use_tc_tiling_on_sc=False, needs_layout_passes=False),
      out_type=jax.ShapeDtypeStruct((SEQ, ET, BT, BLK), jnp.float32),
      scratch_types=[
          pltpu.VMEM((SEQ, BW), jnp.int32),
          pltpu.VMEM((3, BW, D), jnp.float32),
          pltpu.VMEM((3, ET * BLK), jnp.float32),
          pltpu.SemaphoreType.DMA,
          pltpu.SemaphoreType.DMA,
          pltpu.SemaphoreType.DMA,
          pltpu.SemaphoreType.DMA,
          pltpu.SemaphoreType.DMA,
          pltpu.SemaphoreType.DMA,
      ],
  )
  def emb(idx_hbm, table_hbm, out_hbm, idx_v, rows_v, tblk_v,
          gsem0, gsem1, gsem2, osem0, osem1, osem2):
    wid = lax.axis_index("s") * NC + lax.axis_index("c")
    col = pl.multiple_of(wid * BW, BW)
    pltpu.sync_copy(idx_hbm.at[:, pl.ds(col, BW)], idx_v)

    gsems = (gsem0, gsem1, gsem2)
    osems = (osem0, osem1, osem2)
    lane = lax.iota(jnp.int32, 16)
    qs = [(lane + j) & 15 for j in range(16)]
    st_base = [qs[j] * BW + lane for j in range(16)]

    def start_gather(s, b):
      pltpu.make_async_copy(
          table_hbm.at[idx_v.at[s]], rows_v.at[b], gsems[b]).start()

    def wait_gather(s, b):
      pltpu.make_async_copy(
          table_hbm.at[idx_v.at[s]], rows_v.at[b], gsems[b]).wait()

    def transpose(b):
      rows = rows_v.at[b]
      tblk = tblk_v.at[b]

      def g_body(g, carry):
        row_idx = lane + g * 16
        for c in range(D // 16):
          st_off = 16 * c * BW + 16 * g
          vs = [
              plsc.load_gather(rows, [row_idx, qs[j] + 16 * c])
              for j in range(16)
          ]
          for j in range(16):
            plsc.store_scatter(tblk, [st_base[j] + st_off], vs[j])
        return carry

      lax.fori_loop(0, BW // 16, g_body, 0)

    def start_out(s, b):
      for tr in range(ET):
        pltpu.make_async_copy(
            tblk_v.at[b, pl.ds(tr * BLK, BLK)],
            out_hbm.at[s, tr, wid],
            osems[b]).start()

    def wait_out(s, b):
      for tr in range(ET):
        pltpu.make_async_copy(
            tblk_v.at[b, pl.ds(tr * BLK, BLK)],
            out_hbm.at[s, tr, wid],
            osems[b]).wait()

    start_gather(0, 0)
    start_gather(1, 1)
    start_gather(2, 2)

    NB = 3
    STEADY = (SEQ // NB) - 1

    def body(i, carry):
      for b in range(NB):
        s = NB * i + b
        wait_gather(s, b)

        @pl.when(i >= 1)
        def _():
          wait_out(s, b)

        transpose(b)
        start_gather(s + NB, b)
        start_out(s, b)
      return carry

    lax.fori_loop(0, STEADY, body, 0)
    for s in range(NB * STEADY, SEQ):
      b = s % NB
      wait_gather(s, b)
      wait_out(s, b)
      transpose(b)
      if s + NB < SEQ:
        start_gather(s + NB, b)
      start_out(s, b)
    for s in range(SEQ - NB, SEQ):
      wait_out(s, s % NB)

  return emb


_emb = _build()


def kernel(item_inputs, table):
  idx_t = item_inputs.T.astype(jnp.int32)
  out4 = _emb(idx_t, table)
  out = out4.reshape(SEQ, ET, BT, ES, BW).transpose(2, 4, 0, 1, 3)
  return out.reshape(BATCH, SEQ, D)
